# baseline (speedup 1.0000x reference)
import jax
import jax.numpy as jnp
from jax.experimental import pallas as pl

_N = 100000
_DIM = 128
_BLOCK = 20480
_GRID = -(-_N // _BLOCK)
_NPAD = _BLOCK * _GRID
_MROWS = _BLOCK // _DIM


def _fused_body(x_ref, m_ref, wa_ref, o_ref):
    xb = x_ref[...]
    m = m_ref[...]
    m3 = jnp.broadcast_to(m[:, None, :], (_MROWS, _DIM, _DIM))
    mb = jnp.swapaxes(m3, 1, 2).reshape(_BLOCK, _DIM)
    u = jnp.where(mb, xb, 0.0)
    o_ref[...] = (jnp.dot(xb, wa_ref[: _DIM, :], preferred_element_type=jnp.float32)
                  + jnp.dot(u, wa_ref[_DIM :, :], preferred_element_type=jnp.float32))


def kernel(x, mask, deletion_weight):
    mask2 = jnp.pad(mask, (0, _NPAD - _N)).reshape(_NPAD // _DIM, _DIM)
    eye = jnp.eye(_DIM, dtype=jnp.float32)
    wa = jnp.concatenate([eye, deletion_weight - eye], axis=0)
    return pl.pallas_call(
        _fused_body,
        grid=(_GRID,),
        in_specs=[
            pl.BlockSpec((_BLOCK, _DIM), lambda i: (i, 0)),
            pl.BlockSpec((_MROWS, _DIM), lambda i: (i, 0)),
            pl.BlockSpec((2 * _DIM, _DIM), lambda i: (0, 0)),
        ],
        out_specs=pl.BlockSpec((_BLOCK, _DIM), lambda i: (i, 0)),
        out_shape=jax.ShapeDtypeStruct((_N, _DIM), jnp.float32),
    )(x, mask2, wa)


# speedup vs baseline: 1.0088x; 1.0088x over previous
import jax
import jax.numpy as jnp
from jax.experimental import pallas as pl

_N = 100000
_DIM = 128
_BLOCK = 20480
_GRID = -(-_N // _BLOCK)
_NPAD = _BLOCK * _GRID
_MROWS = _BLOCK // _DIM


def _fused_body(x_ref, m_ref, wa_ref, o_ref):
    xb = x_ref[...]
    m = m_ref[...]
    m3 = jnp.broadcast_to(m[:, None, :], (_MROWS, _DIM, _DIM))
    mb = jnp.swapaxes(m3, 1, 2).reshape(_BLOCK, _DIM)
    u = jnp.where(mb, xb, 0.0)
    xa = jnp.concatenate([xb, u], axis=1).astype(jnp.bfloat16)
    o_ref[...] = jnp.dot(xa, wa_ref[...], preferred_element_type=jnp.float32)


def kernel(x, mask, deletion_weight):
    mask2 = jnp.pad(mask, (0, _NPAD - _N)).reshape(_NPAD // _DIM, _DIM)
    eye = jnp.eye(_DIM, dtype=jnp.float32)
    wa = jnp.concatenate([eye, deletion_weight - eye], axis=0).astype(jnp.bfloat16)
    return pl.pallas_call(
        _fused_body,
        grid=(_GRID,),
        in_specs=[
            pl.BlockSpec((_BLOCK, _DIM), lambda i: (i, 0)),
            pl.BlockSpec((_MROWS, _DIM), lambda i: (i, 0)),
            pl.BlockSpec((2 * _DIM, _DIM), lambda i: (0, 0)),
        ],
        out_specs=pl.BlockSpec((_BLOCK, _DIM), lambda i: (i, 0)),
        out_shape=jax.ShapeDtypeStruct((_N, _DIM), jnp.float32),
    )(x, mask2, wa)


# speedup vs baseline: 1.0242x; 1.0153x over previous
import jax
import jax.numpy as jnp
from jax.experimental import pallas as pl

_N = 100000
_DIM = 128
_BLOCK = 20480
_GRID = -(-_N // _BLOCK)
_NPAD = _BLOCK * _GRID
_MROWS = _BLOCK // _DIM


def _fused_body(x_ref, m_ref, wa_ref, o_ref):
    xb = x_ref[...]
    m = m_ref[...]
    m3 = jnp.broadcast_to(m[:, None, :], (_MROWS, _DIM, _DIM))
    mb = jnp.swapaxes(m3, 1, 2).reshape(_BLOCK, _DIM)
    u = jnp.where(mb, xb, 0.0)
    xa = jnp.concatenate([xb, u], axis=1)
    o_ref[...] = jnp.dot(xa, wa_ref[...], preferred_element_type=jnp.float32)


def kernel(x, mask, deletion_weight):
    mask2 = jnp.pad(mask, (0, _NPAD - _N)).reshape(_NPAD // _DIM, _DIM)
    eye = jnp.eye(_DIM, dtype=jnp.float32)
    wa = jnp.concatenate([eye, deletion_weight - eye], axis=0)
    return pl.pallas_call(
        _fused_body,
        grid=(_GRID,),
        in_specs=[
            pl.BlockSpec((_BLOCK, _DIM), lambda i: (i, 0)),
            pl.BlockSpec((_MROWS, _DIM), lambda i: (i, 0)),
            pl.BlockSpec((2 * _DIM, _DIM), lambda i: (0, 0)),
        ],
        out_specs=pl.BlockSpec((_BLOCK, _DIM), lambda i: (i, 0)),
        out_shape=jax.ShapeDtypeStruct((_N, _DIM), jnp.float32),
    )(x, mask2, wa)
